# baseline (speedup 1.0000x reference)
import functools

import jax
import jax.numpy as jnp
from jax import lax
from jax.experimental import pallas as pl
from jax.experimental.pallas import tpu as pltpu
from jax.experimental.pallas import tpu_sc as plsc

NC = 2
NS = 16
NWORK = NC * NS


def _sc_gather(in_table, out_table, i_idx, all_idx, B, E, K):
    b_per = B // NWORK
    r_per = (K * B) // NWORK
    n_ch = 3
    CH = r_per // n_ch
    assert b_per % 8 == 0 and r_per % n_ch == 0 and CH % 8 == 0

    mesh = plsc.VectorSubcoreMesh(core_axis_name="c", subcore_axis_name="s")

    @functools.partial(
        pl.kernel,
        out_type=(
            jax.ShapeDtypeStruct((B, E), jnp.float32),
            jax.ShapeDtypeStruct((K * B, E), jnp.float32),
        ),
        mesh=mesh,
        compiler_params=pltpu.CompilerParams(use_tc_tiling_on_sc=False),
        scratch_types=[
            pltpu.VMEM((b_per,), jnp.int32),
            pltpu.VMEM((b_per, E), jnp.float32),
            pltpu.VMEM((CH,), jnp.int32),
            pltpu.VMEM((CH, E), jnp.float32),
            pltpu.SemaphoreType.DMA,
        ],
    )
    def k(in_hbm, out_hbm, ii_hbm, ai_hbm, inrows_hbm, outrows_hbm,
          iv, irows, av, arows, sem):
        wid = lax.axis_index("s") * NC + lax.axis_index("c")
        base = wid * b_per
        pltpu.sync_copy(ii_hbm.at[pl.ds(base, b_per)], iv)
        pltpu.async_copy(in_hbm.at[iv], irows, sem).wait()
        pltpu.sync_copy(irows, inrows_hbm.at[pl.ds(base, b_per)])
        rbase = wid * r_per
        for c in range(n_ch):
            off = rbase + c * CH
            pltpu.sync_copy(ai_hbm.at[pl.ds(off, CH)], av)
            pltpu.async_copy(out_hbm.at[av], arows, sem).wait()
            pltpu.sync_copy(arows, outrows_hbm.at[pl.ds(off, CH)])

    return k(in_table, out_table, i_idx, all_idx)


def _tc_loss(in_rows, out_rows, B, E, K, W):
    BLK = 2048
    grid = B // BLK

    def body(in_ref, out_ref, o_ref):
        inb = in_ref[...]
        pos = out_ref[0]
        for w in range(1, W):
            pos = pos + out_ref[w]
        neg = out_ref[W]
        for n in range(W + 1, K):
            neg = neg + out_ref[n]
        s_pos = jnp.sum(pos * inb, axis=1)
        s_neg = jnp.sum(neg * inb, axis=1)

        def logsig(x):
            return jnp.minimum(x, 0.0) - jnp.log1p(jnp.exp(-jnp.abs(x)))

        part = jnp.sum(logsig(s_neg) - logsig(s_pos)) * (1.0 / B)

        @pl.when(pl.program_id(0) == 0)
        def _():
            o_ref[0, 0] = 0.0

        o_ref[0, 0] += part

    return pl.pallas_call(
        body,
        grid=(grid,),
        in_specs=[
            pl.BlockSpec((BLK, E), lambda g: (g, 0)),
            pl.BlockSpec((K, BLK, E), lambda g: (0, g, 0)),
        ],
        out_specs=pl.BlockSpec((1, 1), lambda g: (0, 0),
                               memory_space=pltpu.SMEM),
        out_shape=jax.ShapeDtypeStruct((1, 1), jnp.float32),
    )(in_rows, out_rows)


def kernel(i, o, neg, in_table, out_table):
    B = i.shape[0]
    W = o.shape[1]
    N = neg.shape[1]
    K = W + N
    E = in_table.shape[1]
    i32 = i.astype(jnp.int32)
    all_idx = jnp.concatenate(
        [o.T.astype(jnp.int32), neg.T.astype(jnp.int32)], axis=0
    ).reshape(-1)
    in_rows, out_rows = _sc_gather(in_table, out_table, i32, all_idx, B, E, K)
    loss = _tc_loss(in_rows, out_rows.reshape(K, B, E), B, E, K, W)
    return loss[0, 0]
